# Initial kernel scaffold; baseline (speedup 1.0000x reference)
import jax
import jax.numpy as jnp
from jax.experimental import pallas as pl


def kernel(x, edge_index, c1_W1, c1_b1, c1_W2, c1_b2, c2_W1, c2_b1, c2_W2, c2_b2, l1_W, l1_b, l2_W, l2_b):
    raise NotImplementedError("write your pallas kernel here")


# speedup vs baseline: 7.3169x; 7.3169x over previous
import functools

import jax
import jax.numpy as jnp
from jax import lax
from jax.experimental import pallas as pl
from jax.experimental.pallas import tpu as pltpu
from jax.experimental.pallas import tpu_sc as plsc

_NC = 2
_NS = 16
_NW = _NC * _NS
_CH = 128
_ZR = 64


def _sc_mean_agg(D, n_acc, cpw, with_counts):
    mesh = plsc.VectorSubcoreMesh(core_axis_name="c", subcore_axis_name="s",
                                  num_cores=_NC, num_subcores=_NS)
    out_type = [jax.ShapeDtypeStruct((_NC, n_acc, D), jnp.float32)]
    scratch = [
        pltpu.VMEM((cpw, _CH), jnp.int32),
        pltpu.VMEM((cpw, _CH), jnp.int32),
        pltpu.VMEM((2, _CH, D), jnp.float32),
        pltpu.VMEM((_ZR, D), jnp.float32),
        pltpu.SemaphoreType.DMA,
        pltpu.VMEM_SHARED((n_acc, D), jnp.float32),
    ]
    if with_counts:
        out_type.append(jax.ShapeDtypeStruct((_NC, n_acc, 16), jnp.float32))
        scratch += [
            pltpu.VMEM((_CH, 16), jnp.float32),
            pltpu.VMEM((_ZR, 16), jnp.float32),
            pltpu.VMEM_SHARED((n_acc, 16), jnp.float32),
        ]

    def body(h_hbm, src_hbm, dst_hbm, *refs):
        if with_counts:
            (sum_hbm, cnt_hbm, src_v, dst_v, rows_v, zero_v, gsem, acc_sh,
             ones_v, zero16_v, cnt_sh) = refs
        else:
            sum_hbm, src_v, dst_v, rows_v, zero_v, gsem, acc_sh = refs
        cid = lax.axis_index("c")
        sid = lax.axis_index("s")
        wid = sid * _NC + cid

        pltpu.sync_copy(src_hbm.at[pl.ds(wid * cpw, cpw)], src_v)
        pltpu.sync_copy(dst_hbm.at[pl.ds(wid * cpw, cpw)], dst_v)

        zv = jnp.zeros((16,), jnp.float32)

        def zfill(i, _):
            for k in range(D // 16):
                zero_v[i, pl.ds(k * 16, 16)] = zv
            if with_counts:
                zero16_v[i, :] = zv
            return 0

        lax.fori_loop(0, _ZR, zfill, 0)
        if with_counts:
            ov = jnp.ones((16,), jnp.float32)

            def ofill(i, _):
                ones_v[i, :] = ov
                return 0

            lax.fori_loop(0, _CH, ofill, 0)

        rps = n_acc // _NS
        for t in range(rps // _ZR):
            base = sid * rps + t * _ZR
            pltpu.sync_copy(zero_v, acc_sh.at[pl.ds(base, _ZR)])
            if with_counts:
                pltpu.sync_copy(zero16_v, cnt_sh.at[pl.ds(base, _ZR)])
        plsc.subcore_barrier()

        pltpu.async_copy(h_hbm.at[src_v.at[0]], rows_v.at[0], gsem)

        def pair(p, _):
            j0 = p * 2
            for b in range(2):
                jj = j0 + b
                pltpu.make_async_copy(
                    h_hbm.at[src_v.at[0]], rows_v.at[b], gsem).wait()
                nxt = jj + 1

                @pl.when(nxt < cpw)
                def _start():
                    pltpu.async_copy(
                        h_hbm.at[src_v.at[nxt]], rows_v.at[1 - b], gsem)

                pltpu.sync_copy(rows_v.at[b], acc_sh.at[dst_v.at[jj]],
                                add=True)
                if with_counts:
                    pltpu.sync_copy(ones_v, cnt_sh.at[dst_v.at[jj]],
                                    add=True)
            return 0

        lax.fori_loop(0, cpw // 2, pair, 0)
        plsc.subcore_barrier()

        out_base = sid * rps
        pltpu.sync_copy(acc_sh.at[pl.ds(out_base, rps)],
                        sum_hbm.at[cid, pl.ds(out_base, rps)])
        if with_counts:
            pltpu.sync_copy(cnt_sh.at[pl.ds(out_base, rps)],
                            cnt_hbm.at[cid, pl.ds(out_base, rps)])

    return pl.kernel(body, out_type=out_type, mesh=mesh,
                     scratch_types=scratch,
                     compiler_params=pltpu.CompilerParams(
                         use_tc_tiling_on_sc=False))


def kernel(x, edge_index, c1_W1, c1_b1, c1_W2, c1_b2,
           c2_W1, c2_b1, c2_W2, c2_b2, l1_W, l1_b, l2_W, l2_b):
    N, D = x.shape
    E = edge_index.shape[1]
    H1 = c1_W1.shape[1]
    H2 = c2_W1.shape[1]

    cpw = (-(-E // (_CH * _NW)) + 7) // 8 * 8
    n_chunks = cpw * _NW
    e_pad = n_chunks * _CH
    n_acc = (N // (_NS * _ZR) + 1) * (_NS * _ZR)
    scrap = n_acc - N

    src = edge_index[0]
    dst = edge_index[1]
    pad = e_pad - E
    src_p = jnp.concatenate([src, jnp.zeros((pad,), jnp.int32)])
    dst_p = jnp.concatenate(
        [dst, N + (jnp.arange(pad, dtype=jnp.int32) % scrap)])
    src2 = src_p.reshape(n_chunks, _CH)
    dst2 = dst_p.reshape(n_chunks, _CH)

    f32 = jnp.float32

    def tc1(x_ref, w_ref, b_ref, o_ref):
        o_ref[...] = jnp.maximum(
            jnp.dot(x_ref[...], w_ref[...], preferred_element_type=f32)
            + b_ref[...], 0.0)

    h1 = pl.pallas_call(
        tc1, out_shape=jax.ShapeDtypeStruct((N, H1), f32),
    )(x, c1_W1, c1_b1.reshape(1, H1))

    sum1, cnt1 = _sc_mean_agg(H1, n_acc, cpw, True)(h1, src2, dst2)

    def tc2(x_ref, s_ref, c_ref, w2_ref, b2_ref, w3_ref, b3_ref,
            x2_ref, h2_ref):
        s = s_ref[0][:N] + s_ref[1][:N]
        cnt = jnp.max(c_ref[0][:N] + c_ref[1][:N], axis=1, keepdims=True)
        m = s / jnp.maximum(cnt, 1.0)
        a = (jnp.dot(x_ref[...], w2_ref[:D], preferred_element_type=f32)
             + jnp.dot(m, w2_ref[D:], preferred_element_type=f32)
             + b2_ref[...])
        x2 = jnp.maximum(a, 0.0)
        x2_ref[...] = x2
        h2_ref[...] = jnp.maximum(
            jnp.dot(x2, w3_ref[...], preferred_element_type=f32)
            + b3_ref[...], 0.0)

    x2, h2 = pl.pallas_call(
        tc2, out_shape=[jax.ShapeDtypeStruct((N, D), f32),
                        jax.ShapeDtypeStruct((N, H2), f32)],
    )(x, sum1, cnt1, c1_W2, c1_b2.reshape(1, -1), c2_W1,
      c2_b1.reshape(1, H2))

    (sum2,) = _sc_mean_agg(H2, n_acc, cpw, False)(h2, src2, dst2)

    def tc3(x2_ref, s_ref, c_ref, w2_ref, b2_ref, wl1_ref, bl1_ref,
            wl2_ref, bl2_ref, y_ref):
        s = s_ref[0][:N] + s_ref[1][:N]
        cnt = jnp.max(c_ref[0][:N] + c_ref[1][:N], axis=1, keepdims=True)
        m = s / jnp.maximum(cnt, 1.0)
        a = (jnp.dot(x2_ref[...], w2_ref[:D], preferred_element_type=f32)
             + jnp.dot(m, w2_ref[D:], preferred_element_type=f32)
             + b2_ref[...])
        o2 = jnp.maximum(a, 0.0)
        h3 = jnp.maximum(
            jnp.dot(o2, wl1_ref[...], preferred_element_type=f32)
            + bl1_ref[...], 0.0)
        y_ref[...] = (jnp.dot(h3, wl2_ref[...], preferred_element_type=f32)
                      + bl2_ref[...])

    y = pl.pallas_call(
        tc3, out_shape=jax.ShapeDtypeStruct((N, 1), f32),
    )(x2, sum2, cnt1, c2_W2, c2_b2.reshape(1, -1), l1_W,
      l1_b.reshape(1, -1), l2_W, l2_b.reshape(1, -1))
    return y
